# Initial kernel scaffold; baseline (speedup 1.0000x reference)
import jax
import jax.numpy as jnp
from jax.experimental import pallas as pl


def kernel(z_t, ln_g, ln_b, step_table, router_W, router_b, e_fc1_W, e_fc1_b, e_fc2_W, e_fc2_b, s_fc1_W, s_fc1_b, s_fc2_W, s_fc2_b, step_id):
    raise NotImplementedError("write your pallas kernel here")


# speedup vs baseline: 2.1955x; 2.1955x over previous
import functools

import jax
import jax.numpy as jnp
from jax import lax
from jax.experimental import pallas as pl
from jax.experimental.pallas import tpu as pltpu

B = 2048
D = 1024
E = 16
K = 2
SMAX = 32
INNER = 2048

RBT = 256
MBT = 1024


def _ln_f32(z, g, b):
    m = jnp.mean(z, axis=-1, keepdims=True)
    c = z - m
    v = jnp.mean(c * c, axis=-1, keepdims=True)
    return c * lax.rsqrt(v + 1e-5) * g + b


def _router_body(z_ref, step_ref, wz_ref, ws_ref, rb_ref, g_ref, b_ref,
                 u_ref, wts_ref, psum_ref, esum_ref, bl_ref, re_ref):
    t = pl.program_id(0)
    z = z_ref[...]
    c = jnp.sum(ws_ref[...] * step_ref[...], axis=-1, keepdims=True)
    logits = jax.lax.dot_general(z, wz_ref[...], (((1,), (1,)), ((), ())),
                                 preferred_element_type=jnp.float32)
    logits = logits + c.T + rb_ref[...]
    mx = jnp.max(logits, axis=-1, keepdims=True)
    ex = jnp.exp(logits - mx)
    probs = ex / jnp.sum(ex, axis=-1, keepdims=True)

    ids = lax.broadcasted_iota(jnp.int32, probs.shape, 1)
    m1 = jnp.max(probs, axis=-1, keepdims=True)
    i1 = jnp.min(jnp.where(probs == m1, ids, E), axis=-1, keepdims=True)
    masked = jnp.where(ids == i1, -jnp.inf, probs)
    m2 = jnp.max(masked, axis=-1, keepdims=True)
    i2 = jnp.min(jnp.where(masked == m2, ids, E), axis=-1, keepdims=True)
    wts_ref[...] = jnp.where(ids == i1, m1, 0.0) + jnp.where(ids == i2, m2, 0.0)

    ent = -jnp.sum(probs * jnp.log(jnp.clip(probs, 1e-8, None)))
    pblk = jnp.sum(probs, axis=0, keepdims=True)

    @pl.when(t == 0)
    def _():
        psum_ref[...] = jnp.zeros_like(psum_ref)
        esum_ref[...] = jnp.zeros_like(esum_ref)
    psum_ref[...] += pblk
    esum_ref[...] += ent.reshape(1, 1)

    @pl.when(t == pl.num_programs(0) - 1)
    def _():
        avg = psum_ref[...] / B
        dev = avg - (1.0 / E)
        bl_ref[...] = jnp.mean(dev * dev).reshape(1, 1)
        re_ref[...] = esum_ref[...] / B

    u = _ln_f32(z, g_ref[...], b_ref[...])
    u_ref[...] = u.astype(jnp.bfloat16)


def _router_call(z_t, step_row, wz, ws, rb, g, b):
    nt = B // RBT
    return pl.pallas_call(
        _router_body,
        grid=(nt,),
        in_specs=[
            pl.BlockSpec((RBT, D), lambda t: (t, 0)),
            pl.BlockSpec((1, D), lambda t: (0, 0)),
            pl.BlockSpec((E, D), lambda t: (0, 0)),
            pl.BlockSpec((E, D), lambda t: (0, 0)),
            pl.BlockSpec((1, E), lambda t: (0, 0)),
            pl.BlockSpec((1, D), lambda t: (0, 0)),
            pl.BlockSpec((1, D), lambda t: (0, 0)),
        ],
        out_specs=[
            pl.BlockSpec((RBT, D), lambda t: (t, 0)),
            pl.BlockSpec((RBT, E), lambda t: (t, 0)),
            pl.BlockSpec((1, E), lambda t: (0, 0)),
            pl.BlockSpec((1, 1), lambda t: (0, 0)),
            pl.BlockSpec((1, 1), lambda t: (0, 0)),
            pl.BlockSpec((1, 1), lambda t: (0, 0)),
        ],
        out_shape=[
            jax.ShapeDtypeStruct((B, D), jnp.bfloat16),
            jax.ShapeDtypeStruct((B, E), jnp.float32),
            jax.ShapeDtypeStruct((1, E), jnp.float32),
            jax.ShapeDtypeStruct((1, 1), jnp.float32),
            jax.ShapeDtypeStruct((1, 1), jnp.float32),
            jax.ShapeDtypeStruct((1, 1), jnp.float32),
        ],
        compiler_params=pltpu.CompilerParams(
            dimension_semantics=("arbitrary",)),
    )(z_t, step_row, wz, ws, rb, g, b)


def _gelu_exact(x):
    return 0.5 * x * (1.0 + lax.erf(x * 0.7071067811865476))


def _shared_body(u_ref, z_ref, w1_ref, b1_ref, w2_ref, b2_ref, out_ref):
    h = jax.lax.dot_general(u_ref[...], w1_ref[...],
                            (((1,), (1,)), ((), ())),
                            preferred_element_type=jnp.float32)
    h = _gelu_exact(h + b1_ref[...]).astype(jnp.bfloat16)
    y = jax.lax.dot_general(h, w2_ref[...],
                            (((1,), (1,)), ((), ())),
                            preferred_element_type=jnp.float32)
    out_ref[...] = z_ref[...] + y + b2_ref[...]


def _shared_call(u, z_t, w1, b1, w2, b2):
    nt = B // MBT
    return pl.pallas_call(
        _shared_body,
        grid=(nt,),
        in_specs=[
            pl.BlockSpec((MBT, D), lambda t: (t, 0)),
            pl.BlockSpec((MBT, D), lambda t: (t, 0)),
            pl.BlockSpec((INNER, D), lambda t: (0, 0)),
            pl.BlockSpec((1, INNER), lambda t: (0, 0)),
            pl.BlockSpec((D, INNER), lambda t: (0, 0)),
            pl.BlockSpec((1, D), lambda t: (0, 0)),
        ],
        out_specs=pl.BlockSpec((MBT, D), lambda t: (t, 0)),
        out_shape=jax.ShapeDtypeStruct((B, D), jnp.float32),
        compiler_params=pltpu.CompilerParams(
            dimension_semantics=("arbitrary",)),
    )(u, z_t, w1, b1, w2, b2)


def _experts_body(u_ref, base_ref, wts_ref, w1_ref, b1_ref, w2_ref, b2_ref,
                  out_ref):
    e = pl.program_id(1)
    h = jax.lax.dot_general(u_ref[...], w1_ref[0],
                            (((1,), (1,)), ((), ())),
                            preferred_element_type=jnp.float32)
    h = _gelu_exact(h + b1_ref[0]).astype(jnp.bfloat16)
    y = jax.lax.dot_general(h, w2_ref[0],
                            (((1,), (1,)), ((), ())),
                            preferred_element_type=jnp.float32)
    y = y + b2_ref[0]
    wts = wts_ref[...]
    sel = lax.broadcasted_iota(jnp.int32, wts.shape, 1) == e
    w_e = jnp.sum(jnp.where(sel, wts, 0.0), axis=1, keepdims=True)
    contrib = w_e * y

    @pl.when(e == 0)
    def _():
        out_ref[...] = base_ref[...] + contrib

    @pl.when(e != 0)
    def _():
        out_ref[...] += contrib


def _experts_call(u, base, wts, w1, b1, w2, b2):
    nt = B // MBT
    return pl.pallas_call(
        _experts_body,
        grid=(nt, E),
        in_specs=[
            pl.BlockSpec((MBT, D), lambda t, e: (t, 0)),
            pl.BlockSpec((MBT, D), lambda t, e: (t, 0)),
            pl.BlockSpec((MBT, E), lambda t, e: (t, 0)),
            pl.BlockSpec((1, INNER, D), lambda t, e: (e, 0, 0)),
            pl.BlockSpec((1, 1, INNER), lambda t, e: (e, 0, 0)),
            pl.BlockSpec((1, D, INNER), lambda t, e: (e, 0, 0)),
            pl.BlockSpec((1, 1, D), lambda t, e: (e, 0, 0)),
        ],
        out_specs=pl.BlockSpec((MBT, D), lambda t, e: (t, 0)),
        out_shape=jax.ShapeDtypeStruct((B, D), jnp.float32),
        compiler_params=pltpu.CompilerParams(
            dimension_semantics=("arbitrary", "arbitrary")),
    )(u, base, wts, w1, b1, w2, b2)


def kernel(z_t, ln_g, ln_b, step_table, router_W, router_b, e_fc1_W, e_fc1_b,
           e_fc2_W, e_fc2_b, s_fc1_W, s_fc1_b, s_fc2_W, s_fc2_b, step_id):
    sid = jnp.clip(jnp.asarray(step_id, jnp.int32), 0, SMAX - 1)
    step_row = lax.dynamic_slice_in_dim(step_table, sid, 1, 0)
    wz = router_W[:, :D]
    ws = router_W[:, D:]

    u, wts, _psum, _esum, bl, re = _router_call(
        z_t, step_row, wz, ws, router_b.reshape(1, E),
        ln_g.reshape(1, D), ln_b.reshape(1, D))

    base = _shared_call(u, z_t, s_fc1_W.astype(jnp.bfloat16),
                        s_fc1_b.reshape(1, INNER),
                        s_fc2_W.astype(jnp.bfloat16), s_fc2_b.reshape(1, D))

    z_out = _experts_call(u, base, wts, e_fc1_W.astype(jnp.bfloat16),
                          e_fc1_b.reshape(E, 1, INNER),
                          e_fc2_W.astype(jnp.bfloat16),
                          e_fc2_b.reshape(E, 1, D))

    return (z_out, bl.reshape(()), re.reshape(()))
